# Initial kernel scaffold; baseline (speedup 1.0000x reference)
import jax
import jax.numpy as jnp
from jax.experimental import pallas as pl


def kernel(sin_table, temp_idx):
    raise NotImplementedError("write your pallas kernel here")


# speedup vs baseline: 7.0372x; 7.0372x over previous
import functools

import jax
import jax.numpy as jnp
from jax import lax
from jax.experimental import pallas as pl
from jax.experimental.pallas import tpu as pltpu
from jax.experimental.pallas import tpu_sc as plsc

D = 128
BATCH = 4096
SEQ = 200
B = BATCH * SEQ

NC = 2
NS = 16
NW = NC * NS
BPW = B // NW
CH = 128
NCHUNK = BPW // CH

_mesh = plsc.VectorSubcoreMesh(core_axis_name="c", subcore_axis_name="s")


@functools.partial(
    pl.kernel,
    mesh=_mesh,
    out_type=jax.ShapeDtypeStruct((B, D), jnp.float32),
    scratch_types=[
        pltpu.VMEM((NCHUNK, CH), jnp.int32),
        pltpu.VMEM((CH, D), jnp.float32),
        pltpu.SemaphoreType.DMA,
    ],
)
def _gather_kernel(table_hbm, idx_hbm, out_hbm, idx_v, rows_v, sem):
    wid = lax.axis_index("s") * NC + lax.axis_index("c")
    base = wid * BPW
    pltpu.sync_copy(idx_hbm.at[pl.ds(wid * NCHUNK, NCHUNK)], idx_v)

    def body(j, carry):
        pltpu.async_copy(table_hbm.at[idx_v.at[j]], rows_v, sem).wait()
        pltpu.sync_copy(rows_v, out_hbm.at[pl.ds(base + j * CH, CH)])
        return carry

    lax.fori_loop(0, NCHUNK, body, 0)


def kernel(sin_table, temp_idx):
    idx = temp_idx.astype(jnp.int32).reshape(NW * NCHUNK, CH)
    out = _gather_kernel(sin_table, idx)
    return out.reshape(BATCH, SEQ, D)


# speedup vs baseline: 9.8678x; 1.4022x over previous
import functools

import jax
import jax.numpy as jnp
from jax import lax
from jax.experimental import pallas as pl
from jax.experimental.pallas import tpu as pltpu
from jax.experimental.pallas import tpu_sc as plsc

D = 128
BATCH = 4096
SEQ = 200
B = BATCH * SEQ

NC = 2
NS = 16
NW = NC * NS
BPW = B // NW
CH = 128
NCHUNK = BPW // CH
NBUF = 2
GROUP = NBUF * CH
NGRP = BPW // GROUP

_mesh = plsc.VectorSubcoreMesh(core_axis_name="c", subcore_axis_name="s")


@functools.partial(
    pl.kernel,
    mesh=_mesh,
    out_type=jax.ShapeDtypeStruct((B, D), jnp.float32),
    scratch_types=[
        pltpu.VMEM((NCHUNK, CH), jnp.int32),
        pltpu.VMEM((GROUP, D), jnp.float32),
        pltpu.VMEM((GROUP, D), jnp.float32),
        pltpu.SemaphoreType.DMA,
        pltpu.SemaphoreType.DMA,
        pltpu.SemaphoreType.DMA,
        pltpu.SemaphoreType.DMA,
    ],
)
def _gather_kernel(table_hbm, idx_hbm, out_hbm, idx_v, rows_a, rows_b,
                   gsem_a, gsem_b, ssem_a, ssem_b):
    wid = lax.axis_index("s") * NC + lax.axis_index("c")
    base = wid * BPW
    pltpu.sync_copy(idx_hbm.at[pl.ds(wid * NCHUNK, NCHUNK)], idx_v)

    def issue_gathers(g, rows, sem):
        for c in range(NBUF):
            pltpu.async_copy(
                table_hbm.at[idx_v.at[g * NBUF + c]],
                rows.at[pl.ds(c * CH, CH)],
                sem,
            )

    def drain_gathers(rows, sem):
        pltpu.make_async_copy(out_hbm.at[pl.ds(0, GROUP)], rows, sem).wait()

    def issue_scatter(g, rows, sem):
        pltpu.async_copy(rows, out_hbm.at[pl.ds(base + g * GROUP, GROUP)], sem)

    def drain_scatter(g, rows, sem):
        pltpu.make_async_copy(
            rows, out_hbm.at[pl.ds(base + g * GROUP, GROUP)], sem).wait()

    issue_gathers(0, rows_a, gsem_a)

    def body(h, carry):
        a = 2 * h
        drain_gathers(rows_a, gsem_a)
        issue_scatter(a, rows_a, ssem_a)

        @pl.when(h > 0)
        def _():
            drain_scatter(a - 1, rows_b, ssem_b)
        issue_gathers(a + 1, rows_b, gsem_b)

        drain_gathers(rows_b, gsem_b)
        issue_scatter(a + 1, rows_b, ssem_b)

        @pl.when(a + 2 < NGRP)
        def _():
            drain_scatter(a, rows_a, ssem_a)
            issue_gathers(a + 2, rows_a, gsem_a)

        return carry

    lax.fori_loop(0, NGRP // 2, body, 0)

    drain_scatter(NGRP - 2, rows_a, ssem_a)
    drain_scatter(NGRP - 1, rows_b, ssem_b)


def kernel(sin_table, temp_idx):
    idx = temp_idx.astype(jnp.int32).reshape(NW * NCHUNK, CH)
    out = _gather_kernel(sin_table, idx)
    return out.reshape(BATCH, SEQ, D)


# speedup vs baseline: 10.0387x; 1.0173x over previous
import functools

import jax
import jax.numpy as jnp
from jax import lax
from jax.experimental import pallas as pl
from jax.experimental.pallas import tpu as pltpu
from jax.experimental.pallas import tpu_sc as plsc

D = 128
BATCH = 4096
SEQ = 200
B = BATCH * SEQ

NC = 2
NS = 16
NW = NC * NS
BPW = B // NW
CH = 128
NCHUNK = BPW // CH
NB = 5
LOOK = 2

_mesh = plsc.VectorSubcoreMesh(core_axis_name="c", subcore_axis_name="s")


@functools.partial(
    pl.kernel,
    mesh=_mesh,
    out_type=jax.ShapeDtypeStruct((B, D), jnp.float32),
    scratch_types=(
        [pltpu.VMEM((NCHUNK, CH), jnp.int32)]
        + [pltpu.VMEM((CH, D), jnp.float32) for _ in range(NB)]
        + [pltpu.SemaphoreType.DMA for _ in range(2 * NB)]
    ),
)
def _gather_kernel(table_hbm, idx_hbm, out_hbm, idx_v, *scratch):
    rows = scratch[:NB]
    gsem = scratch[NB:2 * NB]
    ssem = scratch[2 * NB:]

    wid = lax.axis_index("s") * NC + lax.axis_index("c")
    base = wid * BPW
    pltpu.sync_copy(idx_hbm.at[pl.ds(wid * NCHUNK, NCHUNK)], idx_v)

    def issue_gather(j, b):
        pltpu.async_copy(table_hbm.at[idx_v.at[j]], rows[b], gsem[b])

    def drain_gather(b):
        pltpu.make_async_copy(out_hbm.at[pl.ds(0, CH)], rows[b], gsem[b]).wait()

    def issue_scatter(j, b):
        pltpu.async_copy(rows[b], out_hbm.at[pl.ds(base + j * CH, CH)], ssem[b])

    def drain_scatter(b):
        pltpu.make_async_copy(
            rows[b], out_hbm.at[pl.ds(0, CH)], ssem[b]).wait()

    for j in range(LOOK):
        issue_gather(j, j)

    def body(g, carry):
        for b in range(NB):
            j = g * NB + b
            drain_gather(b)
            issue_scatter(j, b)
            jj = j + LOOK
            b2 = (b + LOOK) % NB

            @pl.when(jj >= NB)
            def _():
                drain_scatter(b2)

            @pl.when(jj < NCHUNK)
            def _():
                issue_gather(jj, b2)
        return carry

    lax.fori_loop(0, NCHUNK // NB, body, 0)

    for b in range(LOOK, NB):
        drain_scatter(b)


def kernel(sin_table, temp_idx):
    idx = temp_idx.astype(jnp.int32).reshape(NW * NCHUNK, CH)
    out = _gather_kernel(sin_table, idx)
    return out.reshape(BATCH, SEQ, D)


# speedup vs baseline: 16.4224x; 1.6359x over previous
import functools

import jax
import jax.numpy as jnp
from jax import lax
from jax.experimental import pallas as pl
from jax.experimental.pallas import tpu as pltpu
from jax.experimental.pallas import tpu_sc as plsc

D = 128
BATCH = 4096
SEQ = 200
B = BATCH * SEQ
NROWS = 10001

NC = 2
NS = 16
NW = NC * NS
BPW = B // NW
CH = 128
NCHUNK = BPW // CH
NB = 2
LOOK = 1
NIB = NB
ILOOK = NIB

_mesh = plsc.VectorSubcoreMesh(core_axis_name="c", subcore_axis_name="s")


@functools.partial(
    pl.kernel,
    mesh=_mesh,
    out_type=jax.ShapeDtypeStruct((B, D), jnp.float32),
    scratch_types=(
        [pltpu.VMEM_SHARED((NROWS, D), jnp.float32)]
        + [pltpu.VMEM((CH, D), jnp.float32) for _ in range(NB)]
        + [pltpu.VMEM((CH,), jnp.int32) for _ in range(NIB)]
        + [pltpu.SemaphoreType.DMA for _ in range(2 * NB + NIB)]
    ),
)
def _gather_kernel(table_hbm, idx_hbm, out_hbm, table_sp, *scratch):
    rows = scratch[:NB]
    ibuf = scratch[NB:NB + NIB]
    gsem = scratch[NB + NIB:2 * NB + NIB]
    ssem = scratch[2 * NB + NIB:3 * NB + NIB]
    isem = scratch[3 * NB + NIB:]

    sid = lax.axis_index("s")
    wid = sid * NC + lax.axis_index("c")
    base = wid * BPW

    @pl.when(sid == 0)
    def _():
        pltpu.sync_copy(table_hbm, table_sp)

    def issue_idx(j, b):
        pltpu.async_copy(idx_hbm.at[pl.ds(base + j * CH, CH)], ibuf[b], isem[b])

    def drain_idx(b):
        pltpu.make_async_copy(idx_hbm.at[pl.ds(0, CH)], ibuf[b], isem[b]).wait()

    def issue_gather(b):
        pltpu.async_copy(table_sp.at[ibuf[b]], rows[b], gsem[b])

    def drain_gather(b):
        pltpu.make_async_copy(out_hbm.at[pl.ds(0, CH)], rows[b], gsem[b]).wait()

    def issue_scatter(j, b):
        pltpu.async_copy(rows[b], out_hbm.at[pl.ds(base + j * CH, CH)], ssem[b])

    def drain_scatter(b):
        pltpu.make_async_copy(
            rows[b], out_hbm.at[pl.ds(0, CH)], ssem[b]).wait()

    for j in range(ILOOK):
        issue_idx(j, j % NIB)
    plsc.subcore_barrier()
    for j in range(LOOK):
        drain_idx(j % NIB)
        issue_gather(j % NB)

    def body(g, carry):
        for b in range(NB):
            j = g * NB + b
            drain_gather(b)
            issue_scatter(j, b)

            @pl.when(j + ILOOK < NCHUNK)
            def _():
                issue_idx(j + ILOOK, b)

            jj = j + LOOK
            b2 = (b + LOOK) % NB

            @pl.when(jj >= NB)
            def _():
                drain_scatter(b2)

            @pl.when(jj < NCHUNK)
            def _():
                drain_idx((b + LOOK) % NIB)
                issue_gather(b2)
        return carry

    lax.fori_loop(0, NCHUNK // NB, body, 0)

    for b in range(LOOK, NB):
        drain_scatter(b)


def kernel(sin_table, temp_idx):
    idx = temp_idx.astype(jnp.int32).reshape(B)
    out = _gather_kernel(sin_table, idx)
    return out.reshape(BATCH, SEQ, D)


# speedup vs baseline: 17.5431x; 1.0682x over previous
import functools

import jax
import jax.numpy as jnp
from jax import lax
from jax.experimental import pallas as pl
from jax.experimental.pallas import tpu as pltpu
from jax.experimental.pallas import tpu_sc as plsc

D = 128
BATCH = 4096
SEQ = 200
B = BATCH * SEQ
NROWS = 10001

NC = 2
NS = 16
NW = NC * NS
BPW = B // NW
CH = 64
NCHUNK = BPW // CH
NB = 4
LOOK = 2
NIB = NB
ILOOK = NIB

_mesh = plsc.VectorSubcoreMesh(core_axis_name="c", subcore_axis_name="s")


@functools.partial(
    pl.kernel,
    mesh=_mesh,
    out_type=jax.ShapeDtypeStruct((B, D), jnp.float32),
    scratch_types=(
        [pltpu.VMEM_SHARED((NROWS, D), jnp.float32)]
        + [pltpu.VMEM((CH, D), jnp.float32) for _ in range(NB)]
        + [pltpu.VMEM((CH,), jnp.int32) for _ in range(NIB)]
        + [pltpu.SemaphoreType.DMA for _ in range(2 * NB + NIB)]
    ),
)
def _gather_kernel(table_hbm, idx_hbm, out_hbm, table_sp, *scratch):
    rows = scratch[:NB]
    ibuf = scratch[NB:NB + NIB]
    gsem = scratch[NB + NIB:2 * NB + NIB]
    ssem = scratch[2 * NB + NIB:3 * NB + NIB]
    isem = scratch[3 * NB + NIB:]

    sid = lax.axis_index("s")
    wid = sid * NC + lax.axis_index("c")
    base = wid * BPW

    @pl.when(sid == 0)
    def _():
        pltpu.sync_copy(table_hbm, table_sp)

    def issue_idx(j, b):
        pltpu.async_copy(idx_hbm.at[pl.ds(base + j * CH, CH)], ibuf[b], isem[b])

    def drain_idx(b):
        pltpu.make_async_copy(idx_hbm.at[pl.ds(0, CH)], ibuf[b], isem[b]).wait()

    def issue_gather(b):
        pltpu.async_copy(table_sp.at[ibuf[b]], rows[b], gsem[b])

    def drain_gather(b):
        pltpu.make_async_copy(out_hbm.at[pl.ds(0, CH)], rows[b], gsem[b]).wait()

    def issue_scatter(j, b):
        pltpu.async_copy(rows[b], out_hbm.at[pl.ds(base + j * CH, CH)], ssem[b])

    def drain_scatter(b):
        pltpu.make_async_copy(
            rows[b], out_hbm.at[pl.ds(0, CH)], ssem[b]).wait()

    for j in range(ILOOK):
        issue_idx(j, j % NIB)
    plsc.subcore_barrier()
    for j in range(LOOK):
        drain_idx(j % NIB)
        issue_gather(j % NB)

    def body(g, carry):
        for b in range(NB):
            j = g * NB + b
            drain_gather(b)
            issue_scatter(j, b)

            @pl.when(j + ILOOK < NCHUNK)
            def _():
                issue_idx(j + ILOOK, b)

            jj = j + LOOK
            b2 = (b + LOOK) % NB

            @pl.when(jj >= NB)
            def _():
                drain_scatter(b2)

            @pl.when(jj < NCHUNK)
            def _():
                drain_idx((b + LOOK) % NIB)
                issue_gather(b2)
        return carry

    lax.fori_loop(0, NCHUNK // NB, body, 0)

    for b in range(LOOK, NB):
        drain_scatter(b)


def kernel(sin_table, temp_idx):
    idx = temp_idx.astype(jnp.int32).reshape(B)
    out = _gather_kernel(sin_table, idx)
    return out.reshape(BATCH, SEQ, D)


# speedup vs baseline: 17.8117x; 1.0153x over previous
import functools

import jax
import jax.numpy as jnp
from jax import lax
from jax.experimental import pallas as pl
from jax.experimental.pallas import tpu as pltpu
from jax.experimental.pallas import tpu_sc as plsc

D = 128
BATCH = 4096
SEQ = 200
B = BATCH * SEQ
NROWS = 10001

NC = 2
NS = 16
NW = NC * NS
BPW = B // NW
CH = 32
NCHUNK = BPW // CH
NB = 8
LOOK = 4
NIB = NB
ILOOK = NIB

_mesh = plsc.VectorSubcoreMesh(core_axis_name="c", subcore_axis_name="s")


@functools.partial(
    pl.kernel,
    mesh=_mesh,
    out_type=jax.ShapeDtypeStruct((B, D), jnp.float32),
    scratch_types=(
        [pltpu.VMEM_SHARED((NROWS, D), jnp.float32)]
        + [pltpu.VMEM((CH, D), jnp.float32) for _ in range(NB)]
        + [pltpu.VMEM((CH,), jnp.int32) for _ in range(NIB)]
        + [pltpu.SemaphoreType.DMA for _ in range(2 * NB + NIB)]
    ),
)
def _gather_kernel(table_hbm, idx_hbm, out_hbm, table_sp, *scratch):
    rows = scratch[:NB]
    ibuf = scratch[NB:NB + NIB]
    gsem = scratch[NB + NIB:2 * NB + NIB]
    ssem = scratch[2 * NB + NIB:3 * NB + NIB]
    isem = scratch[3 * NB + NIB:]

    sid = lax.axis_index("s")
    wid = sid * NC + lax.axis_index("c")
    base = wid * BPW

    @pl.when(sid == 0)
    def _():
        pltpu.sync_copy(table_hbm, table_sp)

    def issue_idx(j, b):
        pltpu.async_copy(idx_hbm.at[pl.ds(base + j * CH, CH)], ibuf[b], isem[b])

    def drain_idx(b):
        pltpu.make_async_copy(idx_hbm.at[pl.ds(0, CH)], ibuf[b], isem[b]).wait()

    def issue_gather(b):
        pltpu.async_copy(table_sp.at[ibuf[b]], rows[b], gsem[b])

    def drain_gather(b):
        pltpu.make_async_copy(out_hbm.at[pl.ds(0, CH)], rows[b], gsem[b]).wait()

    def issue_scatter(j, b):
        pltpu.async_copy(rows[b], out_hbm.at[pl.ds(base + j * CH, CH)], ssem[b])

    def drain_scatter(b):
        pltpu.make_async_copy(
            rows[b], out_hbm.at[pl.ds(0, CH)], ssem[b]).wait()

    for j in range(ILOOK):
        issue_idx(j, j % NIB)
    plsc.subcore_barrier()
    for j in range(LOOK):
        drain_idx(j % NIB)
        issue_gather(j % NB)

    def body(g, carry):
        for b in range(NB):
            j = g * NB + b
            drain_gather(b)
            issue_scatter(j, b)

            @pl.when(j + ILOOK < NCHUNK)
            def _():
                issue_idx(j + ILOOK, b)

            jj = j + LOOK
            b2 = (b + LOOK) % NB

            @pl.when(jj >= NB)
            def _():
                drain_scatter(b2)

            @pl.when(jj < NCHUNK)
            def _():
                drain_idx((b + LOOK) % NIB)
                issue_gather(b2)
        return carry

    lax.fori_loop(0, NCHUNK // NB, body, 0)

    for b in range(LOOK, NB):
        drain_scatter(b)


def kernel(sin_table, temp_idx):
    idx = temp_idx.astype(jnp.int32).reshape(B)
    out = _gather_kernel(sin_table, idx)
    return out.reshape(BATCH, SEQ, D)
